# baseline (speedup 1.0000x reference)
import functools

import jax
import jax.numpy as jnp
from jax import lax
from jax.experimental import pallas as pl
from jax.experimental.pallas import tpu as pltpu
from jax.experimental.pallas import tpu_sc as plsc

_B, _S, _D = 4, 4096, 2048
_NC, _NS = 2, 16
_NW = _NC * _NS
_DCHUNKS = _NW // _B
_DW = _D // _DCHUNKS
_L = 16
_JV = _DW // _L
_R = 128
_NCHUNK = _S // _R
_NBUF = 3

_mesh = plsc.VectorSubcoreMesh(core_axis_name="c", subcore_axis_name="s")


@functools.partial(
    pl.kernel,
    out_type=jax.ShapeDtypeStruct((_B, _S, _D), jnp.float32),
    mesh=_mesh,
    scratch_types=[
        *[pltpu.VMEM((_R, _DW), jnp.float32) for _ in range(_NBUF)],
        *[pltpu.SemaphoreType.DMA for _ in range(2 * _NBUF)],
    ],
)
def _sc_cumsum(x_hbm, o_hbm, *scratch):
    bufs = scratch[:_NBUF]
    isems = scratch[_NBUF : 2 * _NBUF]
    osems = scratch[2 * _NBUF :]
    wid = lax.axis_index("s") * _NC + lax.axis_index("c")
    b = wid // _DCHUNKS
    d0 = (wid % _DCHUNKS) * _DW

    def load(g):
        return pltpu.make_async_copy(
            x_hbm.at[b, pl.ds(g * _R, _R), pl.ds(d0, _DW)],
            bufs[g % _NBUF],
            isems[g % _NBUF],
        )

    def store(g):
        return pltpu.make_async_copy(
            bufs[g % _NBUF],
            o_hbm.at[b, pl.ds(g * _R, _R), pl.ds(d0, _DW)],
            osems[g % _NBUF],
        )

    load(0).start()
    load(1).start()

    carry = tuple(jnp.zeros((_L,), jnp.float32) for _ in range(_JV))
    for g in range(_NCHUNK):
        if g + 2 < _NCHUNK:
            if g >= 1:
                store(g - 1).wait()
            load(g + 2).start()
        load(g).wait()
        buf = bufs[g % _NBUF]

        def rows_body(i, c):
            for rr in range(2):
                r = 2 * i + rr
                new = []
                for j in range(_JV):
                    cj = c[j] + buf[r, pl.ds(j * _L, _L)]
                    buf[r, pl.ds(j * _L, _L)] = cj
                    new.append(cj)
                c = tuple(new)
            return c

        carry = lax.fori_loop(0, _R // 2, rows_body, carry)
        store(g).start()

    store(_NCHUNK - 3).wait()
    store(_NCHUNK - 2).wait()
    store(_NCHUNK - 1).wait()


def kernel(input):
    return _sc_cumsum(input)
